# baseline (speedup 1.0000x reference)
import functools

import jax
import jax.numpy as jnp
from jax import lax
from jax.experimental import pallas as pl
from jax.experimental.pallas import tpu as pltpu
from jax.experimental.pallas import tpu_sc as plsc

B = 1024
NUM_CLASS = 1000
K = 16
C = 512
PRE = 6
SUF = 55
T = PRE + K + SUF
MASK_PROB = 0.5

NC, NS, L = 2, 16, 16
NW = NC * NS
BPW = B // NW
G = 4
NG = BPW // G
CL = C // L

_mesh = plsc.VectorSubcoreMesh(
    core_axis_name="c", subcore_axis_name="s", num_cores=NC, num_subcores=NS
)


@functools.partial(
    pl.kernel,
    out_type=jax.ShapeDtypeStruct((B, T * C), jnp.float32),
    mesh=_mesh,
    scratch_types=[
        pltpu.VMEM((NG * 8,), jnp.int32),
        pltpu.VMEM((BPW,), jnp.float32),
        pltpu.VMEM((C,), jnp.float32),
        pltpu.VMEM((G, K * C), jnp.float32),
        pltpu.VMEM((G, K * C), jnp.float32),
        pltpu.VMEM((G, PRE * C), jnp.float32),
        pltpu.VMEM((1, SUF * C), jnp.float32),
        pltpu.SemaphoreType.DMA,
        pltpu.SemaphoreType.DMA,
    ],
)
def _sc_prompts(labels_hbm, ctx_hbm, pre_hbm, suf_hbm, dir_hbm, mask_hbm,
                out_hbm, idx_v, msk_v, d_v, ctx0, ctx1, pre_v, suf_v,
                gsem, osem):
    wid = lax.axis_index("s") * NC + lax.axis_index("c")
    base = wid * BPW

    pltpu.sync_copy(labels_hbm.at[pl.ds(wid * (NG * 8), NG * 8)], idx_v)

    bufs = (ctx0, ctx1)

    def start_gather(g):
        return pltpu.async_copy(
            ctx_hbm.at[idx_v.at[pl.ds(g * 8, G)]], bufs[g % 2], gsem)

    gather_descs = {0: start_gather(0)}

    pltpu.sync_copy(mask_hbm.at[pl.ds(base, BPW)], msk_v)
    pltpu.sync_copy(dir_hbm, d_v)
    for g in range(G):
        pltpu.sync_copy(pre_hbm, pre_v.at[pl.ds(g, 1)])
    pltpu.sync_copy(suf_hbm, suf_v)

    lanes = lax.iota(jnp.int32, L)
    zeros = jnp.zeros((L,), jnp.float32)
    pending = {}

    _dnums = lax.GatherDimensionNumbers(
        offset_dims=(), collapsed_slice_dims=(0,), start_index_map=(0,))

    def lane_take(x, idx):
        return lax.gather(x, idx[:, None], _dnums, slice_sizes=(1,),
                          mode=lax.GatherScatterMode.PROMISE_IN_BOUNDS)

    mask_lo = msk_v[pl.ds(0, L)]
    mask_hi = msk_v[pl.ds(L, L)]

    for g in range(NG):
        buf = bufs[g % 2]
        gather_descs.pop(g).wait()
        if g + 1 < NG:
            if g - 1 in pending:
                for dsc in pending.pop(g - 1):
                    dsc.wait()
            gather_descs[g + 1] = start_gather(g + 1)

        @pl.loop(0, G)
        def _sample(j):
            i = g * G + j
            mchunk = jnp.where(i < L, mask_lo, mask_hi)
            m_splat = lane_take(mchunk, jnp.broadcast_to(i & (L - 1), (L,)))

            @pl.loop(0, K)
            def _row(r):
                def dot_body(c2, acc):
                    return acc + (buf[j, pl.ds(r * C + c2 * L, L)]
                                  * d_v[pl.ds(c2 * L, L)])
                acc = lax.fori_loop(0, CL, dot_body, zeros, unroll=8)
                for h in (8, 4, 2, 1):
                    acc = acc + lane_take(acc, lanes ^ h)
                s = m_splat * acc

                @pl.loop(0, CL, unroll=8)
                def _upd(c2):
                    off = r * C + c2 * L
                    buf[j, pl.ds(off, L)] = (
                        buf[j, pl.ds(off, L)] - s * d_v[pl.ds(c2 * L, L)])

        b0 = base + g * G
        outs = [
            pltpu.async_copy(
                pre_v, out_hbm.at[pl.ds(b0, G), pl.ds(0, PRE * C)], osem),
            pltpu.async_copy(
                buf, out_hbm.at[pl.ds(b0, G), pl.ds(PRE * C, K * C)], osem),
        ]
        for j in range(G):
            outs.append(pltpu.async_copy(
                suf_v,
                out_hbm.at[pl.ds(b0 + j, 1), pl.ds((PRE + K) * C, SUF * C)],
                osem))
        pending[g] = outs

    for g in sorted(pending):
        for dsc in pending[g]:
            dsc.wait()


def kernel(labels, cls_ctx, token_prefix, token_suffix, cloth_direction):
    labels_i = labels.astype(jnp.int32)
    lab_pad = jnp.zeros((NW, NG, 8), jnp.int32)
    lab_pad = lab_pad.at[:, :, :G].set(labels_i.reshape(NW, NG, G))
    lab_flat = lab_pad.reshape(NW * NG * 8)
    ctx_flat = cls_ctx.reshape(NUM_CLASS, K * C)
    pre_flat = token_prefix.reshape(1, PRE * C)
    suf_flat = token_suffix.reshape(1, SUF * C)
    d_flat = cloth_direction.reshape(C)
    mask = (jax.random.uniform(jax.random.key(42), (B,)) < MASK_PROB)
    mask = mask.astype(jnp.float32)
    out = _sc_prompts(lab_flat, ctx_flat, pre_flat, suf_flat, d_flat, mask)
    return out.reshape(B, T, C)


# speedup vs baseline: 1.8216x; 1.8216x over previous
import functools

import jax
import jax.numpy as jnp
from jax import lax
from jax.experimental import pallas as pl
from jax.experimental.pallas import tpu as pltpu
from jax.experimental.pallas import tpu_sc as plsc

B = 1024
NUM_CLASS = 1000
K = 16
C = 512
PRE = 6
SUF = 55
T = PRE + K + SUF
MASK_PROB = 0.5

NC, NS, L = 2, 16, 16
NW = NC * NS
BPW = B // NW
CL = C // L

_mesh = plsc.VectorSubcoreMesh(
    core_axis_name="c", subcore_axis_name="s", num_cores=NC, num_subcores=NS
)


@functools.partial(
    pl.kernel,
    out_type=jax.ShapeDtypeStruct((B, T, C), jnp.float32),
    mesh=_mesh,
    scratch_types=[
        pltpu.VMEM((BPW * 8,), jnp.int32),
        pltpu.VMEM((BPW,), jnp.float32),
        pltpu.VMEM((1, C), jnp.float32),
        pltpu.VMEM((1, K, C), jnp.float32),
        pltpu.VMEM((1, K, C), jnp.float32),
        pltpu.VMEM((1, T, C), jnp.float32),
        pltpu.VMEM((1, T, C), jnp.float32),
        pltpu.VMEM((1, SUF, C), jnp.float32),
        pltpu.SemaphoreType.DMA,
        pltpu.SemaphoreType.DMA,
    ],
)
def _sc_prompts(labels_hbm, ctx_hbm, pre_hbm, suf_hbm, dir_hbm, mask_hbm,
                out_hbm, idx_v, msk_v, d_v, cb0, cb1, rb0, rb1, sbuf,
                gsem, osem):
    wid = lax.axis_index("s") * NC + lax.axis_index("c")
    base = wid * BPW

    pltpu.sync_copy(labels_hbm.at[pl.ds(wid * (BPW * 8), BPW * 8)], idx_v)

    pltpu.async_copy(ctx_hbm.at[idx_v.at[pl.ds(0, 1)]], cb0, gsem)

    pltpu.sync_copy(mask_hbm.at[pl.ds(base, BPW)], msk_v)
    pltpu.sync_copy(dir_hbm, d_v)
    pltpu.sync_copy(pre_hbm, rb0.at[pl.ds(0, 1), pl.ds(0, PRE), :])
    pltpu.sync_copy(pre_hbm, rb1.at[pl.ds(0, 1), pl.ds(0, PRE), :])
    pltpu.sync_copy(suf_hbm, sbuf)

    @pl.loop(0, SUF)
    def _suf_row(u):
        @pl.loop(0, CL, unroll=8)
        def _suf_chunk(c2):
            v = sbuf[0, u, pl.ds(c2 * L, L)]
            rb0[0, PRE + K + u, pl.ds(c2 * L, L)] = v
            rb1[0, PRE + K + u, pl.ds(c2 * L, L)] = v

    lanes = lax.iota(jnp.int32, L)
    zeros = jnp.zeros((L,), jnp.float32)

    _dnums = lax.GatherDimensionNumbers(
        offset_dims=(), collapsed_slice_dims=(0,), start_index_map=(0,))

    def lane_take(x, idx):
        return lax.gather(x, idx[:, None], _dnums, slice_sizes=(1,),
                          mode=lax.GatherScatterMode.PROMISE_IN_BOUNDS)

    mask_lo = msk_v[pl.ds(0, L)]
    mask_hi = msk_v[pl.ds(L, L)]

    def idx_slice(i):
        return idx_v.at[pl.ds(pl.multiple_of(i * 8, 8), 1)]

    def out_wait():
        pltpu.make_async_copy(
            rb0, out_hbm.at[pl.ds(base, 1), :, :], osem).wait()

    def handle(i, cb, cbn, rb, has_next):
        pltpu.make_async_copy(ctx_hbm.at[idx_slice(i)], cb, gsem).wait()

        @pl.when(has_next)
        def _():
            pltpu.async_copy(ctx_hbm.at[idx_slice(i + 1)], cbn, gsem)

        @pl.when(i >= 2)
        def _():
            out_wait()

        mchunk = jnp.where(i < L, mask_lo, mask_hi)
        m_splat = lane_take(mchunk, jnp.broadcast_to(i & (L - 1), (L,)))

        @pl.loop(0, K)
        def _row(r):
            def dot_body(c2, acc):
                return acc + (cb[0, r, pl.ds(c2 * L, L)]
                              * d_v[0, pl.ds(c2 * L, L)])
            acc = lax.fori_loop(0, CL, dot_body, zeros, unroll=8)
            for h in (8, 4, 2, 1):
                acc = acc + lane_take(acc, lanes ^ h)
            s = m_splat * acc

            @pl.loop(0, CL, unroll=8)
            def _upd(c2):
                rb[0, PRE + r, pl.ds(c2 * L, L)] = (
                    cb[0, r, pl.ds(c2 * L, L)] - s * d_v[0, pl.ds(c2 * L, L)])

        pltpu.async_copy(rb, out_hbm.at[pl.ds(base + i, 1), :, :], osem)

    @pl.loop(0, BPW // 2)
    def _pair(g):
        a = 2 * g
        handle(a, cb0, cb1, rb0, a + 1 < BPW)
        handle(a + 1, cb1, cb0, rb1, a + 2 < BPW)

    out_wait()
    out_wait()


def kernel(labels, cls_ctx, token_prefix, token_suffix, cloth_direction):
    labels_i = labels.astype(jnp.int32)
    lab_pad = jnp.zeros((B, 8), jnp.int32)
    lab_pad = lab_pad.at[:, 0].set(labels_i)
    lab_flat = lab_pad.reshape(B * 8)
    mask = (jax.random.uniform(jax.random.key(42), (B,)) < MASK_PROB)
    mask = mask.astype(jnp.float32)
    return _sc_prompts(lab_flat, cls_ctx, token_prefix, token_suffix,
                       cloth_direction, mask)


# speedup vs baseline: 2.6163x; 1.4363x over previous
import functools

import jax
import jax.numpy as jnp
from jax import lax
from jax.experimental import pallas as pl
from jax.experimental.pallas import tpu as pltpu
from jax.experimental.pallas import tpu_sc as plsc

B = 1024
NUM_CLASS = 1000
K = 16
C = 512
PRE = 6
SUF = 55
T = PRE + K + SUF
NCONST = PRE + SUF
MASK_PROB = 0.5

NC, NS, L = 2, 16, 16
NW = NC * NS
BPW = B // NW
CL = C // L
REP = 16

_mesh = plsc.VectorSubcoreMesh(
    core_axis_name="c", subcore_axis_name="s", num_cores=NC, num_subcores=NS
)


@functools.partial(
    pl.kernel,
    out_type=jax.ShapeDtypeStruct((T, B, C), jnp.float32),
    mesh=_mesh,
    scratch_types=[
        pltpu.VMEM((BPW,), jnp.int32),
        pltpu.VMEM((BPW,), jnp.int32),
        pltpu.VMEM((BPW,), jnp.int32),
        pltpu.VMEM((BPW,), jnp.float32),
        pltpu.VMEM((1, C), jnp.float32),
        pltpu.VMEM((BPW, C), jnp.float32),
        pltpu.VMEM((BPW, C), jnp.float32),
        pltpu.VMEM((1, BPW, C), jnp.float32),
        pltpu.VMEM((1, BPW, C), jnp.float32),
        pltpu.VMEM((1, PRE + 2 + SUF, C), jnp.float32),
        pltpu.VMEM((1, REP, C), jnp.float32),
        pltpu.VMEM((1, REP, C), jnp.float32),
        pltpu.SemaphoreType.DMA,
        pltpu.SemaphoreType.DMA,
        pltpu.SemaphoreType.DMA,
    ],
)
def _sc_prompts(labm_hbm, ctx_hbm, pre_hbm, suf_hbm, dir_hbm, mask_hbm,
                out_hbm, lab_v, ix0, ix1, msk_v, d_v, gr0, gr1, ro0, ro1,
                cbuf, rp0, rp1, gsem, osem, csem):
    wid = lax.axis_index("s") * NC + lax.axis_index("c")
    base = wid * BPW

    pltpu.sync_copy(labm_hbm.at[pl.ds(pl.multiple_of(base, 8), BPW)], lab_v)

    lanes = lax.iota(jnp.int32, L)
    zeros = jnp.zeros((L,), jnp.float32)

    ixs = (ix0, ix1)
    grs = (gr0, gr1)
    ros = (ro0, ro1)
    rps = (rp0, rp1)

    def set_row_indices(r, ix):
        ix[pl.ds(0, L)] = lab_v[pl.ds(0, L)] + r
        ix[pl.ds(L, L)] = lab_v[pl.ds(L, L)] + r

    set_row_indices(0, ix0)
    pltpu.async_copy(ctx_hbm.at[ix0], gr0, gsem)

    pltpu.sync_copy(mask_hbm.at[pl.ds(pl.multiple_of(base, 8), BPW)], msk_v)
    pltpu.sync_copy(dir_hbm, d_v)
    pltpu.sync_copy(pre_hbm, cbuf.at[pl.ds(0, 1), pl.ds(0, PRE), :])
    pltpu.sync_copy(suf_hbm, cbuf.at[pl.ds(0, 1), pl.ds(PRE + 2, SUF), :])

    for q, rp in enumerate(rps):
        cr = wid + NW * q

        @pl.when(cr < NCONST)
        def _():
            u = jnp.where(cr < PRE, cr, cr + 2)
            t = jnp.where(cr < PRE, cr, K + cr)

            @pl.loop(0, REP)
            def _rep_s(s2):
                @pl.loop(0, CL, unroll=8)
                def _rep_c(c2):
                    rp[0, s2, pl.ds(c2 * L, L)] = cbuf[0, u, pl.ds(c2 * L, L)]

            @pl.loop(0, B // REP)
            def _fire(jb):
                pltpu.async_copy(
                    rp,
                    out_hbm.at[pl.ds(t, 1),
                               pl.ds(pl.multiple_of(jb * REP, 8), REP), :],
                    csem)

    _dnums = lax.GatherDimensionNumbers(
        offset_dims=(), collapsed_slice_dims=(0,), start_index_map=(0,))

    def lane_take(x, idx):
        return lax.gather(x, idx[:, None], _dnums, slice_sizes=(1,),
                          mode=lax.GatherScatterMode.PROMISE_IN_BOUNDS)

    mask_lo = msk_v[pl.ds(0, L)]
    mask_hi = msk_v[pl.ds(L, L)]

    def handle(r, ix, ixn, gr, grn, ro, has_next, drain_out):
        pltpu.make_async_copy(ctx_hbm.at[ix], gr, gsem).wait()

        @pl.when(has_next)
        def _():
            pltpu.async_copy(ctx_hbm.at[ixn], grn, gsem)

        @pl.when(drain_out)
        def _():
            pltpu.make_async_copy(
                ro, out_hbm.at[pl.ds(PRE, 1),
                               pl.ds(pl.multiple_of(base, 8), BPW), :],
                osem).wait()

        @pl.loop(0, BPW)
        def _sample(s):
            mchunk = jnp.where(s < L, mask_lo, mask_hi)
            m_splat = lane_take(mchunk, jnp.broadcast_to(s & (L - 1), (L,)))

            def dot_body(c2, acc):
                return acc + gr[s, pl.ds(c2 * L, L)] * d_v[0, pl.ds(c2 * L, L)]
            acc = lax.fori_loop(0, CL, dot_body, zeros, unroll=8)
            for h in (8, 4, 2, 1):
                acc = acc + lane_take(acc, lanes ^ h)
            sv = m_splat * acc

            @pl.loop(0, CL, unroll=8)
            def _upd(c2):
                ro[0, s, pl.ds(c2 * L, L)] = (
                    gr[s, pl.ds(c2 * L, L)] - sv * d_v[0, pl.ds(c2 * L, L)])

        pltpu.async_copy(
            ro, out_hbm.at[pl.ds(PRE + r, 1),
                           pl.ds(pl.multiple_of(base, 8), BPW), :], osem)

    @pl.loop(0, K // 2)
    def _pair(g):
        a = 2 * g

        @pl.when(a + 1 < K)
        def _():
            set_row_indices(a + 1, ix1)
        handle(a, ix0, ix1, gr0, gr1, ro0, a + 1 < K, a >= 2)

        @pl.when(a + 2 < K)
        def _():
            set_row_indices(a + 2, ix0)
        handle(a + 1, ix1, ix0, gr1, gr0, ro1, a + 2 < K, a >= 2)

    def out_drain():
        pltpu.make_async_copy(
            ro0, out_hbm.at[pl.ds(PRE, 1),
                            pl.ds(pl.multiple_of(base, 8), BPW), :],
            osem).wait()

    out_drain()
    out_drain()

    for q, rp in enumerate(rps):
        cr = wid + NW * q

        @pl.when(cr < NCONST)
        def _():
            @pl.loop(0, B // REP)
            def _drain(jb):
                pltpu.make_async_copy(
                    rp, out_hbm.at[pl.ds(0, 1), pl.ds(0, REP), :],
                    csem).wait()


def kernel(labels, cls_ctx, token_prefix, token_suffix, cloth_direction):
    labm = labels.astype(jnp.int32) * K
    ctx_rows = cls_ctx.reshape(NUM_CLASS * K, C)
    mask = (jax.random.uniform(jax.random.key(42), (B,)) < MASK_PROB)
    mask = mask.astype(jnp.float32)
    out_t = _sc_prompts(labm, ctx_rows, token_prefix, token_suffix,
                        cloth_direction, mask)
    return jnp.transpose(out_t, (1, 0, 2))
